# Initial kernel scaffold; baseline (speedup 1.0000x reference)
import jax
import jax.numpy as jnp
from jax.experimental import pallas as pl


def kernel(input, Wg, W1, W2):
    raise NotImplementedError("write your pallas kernel here")


# speedup vs baseline: 1.2926x; 1.2926x over previous
import functools

import jax
import jax.numpy as jnp
from jax import lax
from jax.experimental import pallas as pl
from jax.experimental.pallas import tpu as pltpu
from jax.experimental.pallas import tpu_sc as plsc

NUM_E = 8
TOPK = 2
T = 2048
D = 2048
FF = 2048
TB = 512
GB = 256
NROWS = T * TOPK
NB = NROWS // GB
NI = NB + NUM_E - 1

_INTERPRET = False


def _router_body(x_ref, wg_ref, e1_ref, e2_ref, r1_ref, r2_ref,
                 w1_ref, w2_ref, cnt_ref, carry):
    i = pl.program_id(0)

    @pl.when(i == 0)
    def _():
        carry[...] = jnp.zeros_like(carry)

    x = x_ref[...]
    wg = wg_ref[...]
    logits = jnp.dot(x, wg, preferred_element_type=jnp.float32)
    lane = lax.broadcasted_iota(jnp.int32, (TB, 128), 1)
    valid = lane < NUM_E
    neg = jnp.float32(-1e30)
    lv = jnp.where(valid, logits, neg)
    m1 = jnp.max(lv, axis=1, keepdims=True)
    e1 = jnp.min(jnp.where((lv == m1) & valid, lane, 127), axis=1,
                 keepdims=True)
    lv2 = jnp.where(lane == e1, neg, lv)
    m2 = jnp.max(lv2, axis=1, keepdims=True)
    e2 = jnp.min(jnp.where((lv2 == m2) & valid, lane, 127), axis=1,
                 keepdims=True)
    w1 = jax.nn.sigmoid(m1 - m2)
    w2 = 1.0 - w1

    oh1 = (lane == e1).astype(jnp.float32)
    oh2 = (lane == e2).astype(jnp.float32)
    ri = lax.broadcasted_iota(jnp.int32, (TB, TB), 0)
    ci = lax.broadcasted_iota(jnp.int32, (TB, TB), 1)
    strict_lower = (ci < ri).astype(jnp.float32)
    c1 = jnp.dot(strict_lower, oh1, preferred_element_type=jnp.float32)
    c2 = (jnp.dot(strict_lower, oh2, preferred_element_type=jnp.float32)
          + jnp.sum(oh1, axis=0, keepdims=True))
    carry_v = carry[...]
    r1 = jnp.sum((c1 + carry_v) * oh1, axis=1, keepdims=True)
    r2 = jnp.sum((c2 + carry_v) * oh2, axis=1, keepdims=True)
    carry[...] = carry_v + jnp.sum(oh1 + oh2, axis=0, keepdims=True)

    e1_ref[...] = e1
    e2_ref[...] = e2
    r1_ref[...] = r1.astype(jnp.int32)
    r2_ref[...] = r2.astype(jnp.int32)
    w1_ref[...] = w1
    w2_ref[...] = w2
    cnt_ref[...] = carry[...]


def _run_router(x, wg_pad):
    col = lambda dt: jax.ShapeDtypeStruct((T, 1), dt)
    outs = [col(jnp.int32), col(jnp.int32), col(jnp.int32), col(jnp.int32),
            col(jnp.float32), col(jnp.float32),
            jax.ShapeDtypeStruct((1, 128), jnp.float32)]
    colspec = pl.BlockSpec((TB, 1), lambda i: (i, 0))
    return pl.pallas_call(
        _router_body,
        grid=(T // TB,),
        in_specs=[pl.BlockSpec((TB, D), lambda i: (i, 0)),
                  pl.BlockSpec((D, 128), lambda i: (0, 0))],
        out_specs=[colspec] * 6 + [pl.BlockSpec((1, 128), lambda i: (0, 0))],
        out_shape=outs,
        scratch_shapes=[pltpu.VMEM((1, 128), jnp.float32)],
        interpret=_INTERPRET,
    )(x, wg_pad)


_SC_NC = 2
_SC_NS = 16
_SC_NW = _SC_NC * _SC_NS
TPW = T // _SC_NW
CH = 16
NCHUNK = TPW // CH


def _dispatch_sc(x, e1, e2, r1, r2, off16):
    @functools.partial(
        pl.kernel,
        mesh=plsc.VectorSubcoreMesh(core_axis_name="c", subcore_axis_name="s"),
        out_type=[jax.ShapeDtypeStruct((NROWS, D), jnp.float32),
                  jax.ShapeDtypeStruct((T,), jnp.int32),
                  jax.ShapeDtypeStruct((T,), jnp.int32)],
        compiler_params=pltpu.CompilerParams(needs_layout_passes=False),
        scratch_types=[
            pltpu.VMEM((16,), jnp.int32),
            pltpu.VMEM((TPW,), jnp.int32),
            pltpu.VMEM((TPW,), jnp.int32),
            pltpu.VMEM((TPW,), jnp.int32),
            pltpu.VMEM((TPW,), jnp.int32),
            pltpu.VMEM((TPW,), jnp.int32),
            pltpu.VMEM((TPW,), jnp.int32),
            pltpu.VMEM((2 * NCHUNK, CH), jnp.int32),
            pltpu.VMEM((CH, D), jnp.float32),
            pltpu.SemaphoreType.DMA,
        ],
    )
    def k(x_hbm, e1_hbm, e2_hbm, r1_hbm, r2_hbm, off_hbm,
          xs_hbm, p1_hbm, p2_hbm,
          off_v, e1_v, e2_v, r1_v, r2_v, p1_v, p2_v, sidx, rows_v,
          sem):
        wid = lax.axis_index("s") * _SC_NC + lax.axis_index("c")
        base = wid * TPW
        pltpu.sync_copy(off_hbm, off_v)
        pltpu.sync_copy(e1_hbm.at[pl.ds(base, TPW)], e1_v)
        pltpu.sync_copy(e2_hbm.at[pl.ds(base, TPW)], e2_v)
        pltpu.sync_copy(r1_hbm.at[pl.ds(base, TPW)], r1_v)
        pltpu.sync_copy(r2_hbm.at[pl.ds(base, TPW)], r2_v)
        for c in range(NCHUNK):
            p1 = plsc.load_gather(off_v, [e1_v[pl.ds(c * CH, CH)]]) \
                + r1_v[pl.ds(c * CH, CH)]
            p2 = plsc.load_gather(off_v, [e2_v[pl.ds(c * CH, CH)]]) \
                + r2_v[pl.ds(c * CH, CH)]
            p1_v[pl.ds(c * CH, CH)] = p1
            p2_v[pl.ds(c * CH, CH)] = p2
            sidx[2 * c] = p1
            sidx[2 * c + 1] = p2
            pltpu.sync_copy(x_hbm.at[pl.ds(base + c * CH, CH)], rows_v)
            cp1 = pltpu.async_copy(rows_v, xs_hbm.at[sidx.at[2 * c]], sem)
            cp2 = pltpu.async_copy(rows_v, xs_hbm.at[sidx.at[2 * c + 1]], sem)
            cp1.wait()
            cp2.wait()
        pltpu.sync_copy(p1_v, p1_hbm.at[pl.ds(base, TPW)])
        pltpu.sync_copy(p2_v, p2_hbm.at[pl.ds(base, TPW)])

    return k(x, e1, e2, r1, r2, off16)


def _combine_sc(res, p1, p2, w1, w2):
    @functools.partial(
        pl.kernel,
        mesh=plsc.VectorSubcoreMesh(core_axis_name="c", subcore_axis_name="s"),
        out_type=jax.ShapeDtypeStruct((T, D), jnp.float32),
        compiler_params=pltpu.CompilerParams(needs_layout_passes=False),
        scratch_types=[
            pltpu.VMEM((TPW,), jnp.int32),
            pltpu.VMEM((TPW,), jnp.int32),
            pltpu.VMEM((TPW,), jnp.float32),
            pltpu.VMEM((TPW,), jnp.float32),
            pltpu.VMEM((2 * NCHUNK, CH), jnp.int32),
            pltpu.VMEM((CH, D), jnp.float32),
            pltpu.VMEM((CH, D), jnp.float32),
            pltpu.VMEM((CH, D), jnp.float32),
            pltpu.SemaphoreType.DMA,
        ],
    )
    def k(res_hbm, p1_hbm, p2_hbm, w1_hbm, w2_hbm, out_hbm,
          p1_v, p2_v, w1_v, w2_v, sidx, buf_a, buf_b, buf_o, sem):
        wid = lax.axis_index("s") * _SC_NC + lax.axis_index("c")
        base = wid * TPW
        pltpu.sync_copy(p1_hbm.at[pl.ds(base, TPW)], p1_v)
        pltpu.sync_copy(p2_hbm.at[pl.ds(base, TPW)], p2_v)
        pltpu.sync_copy(w1_hbm.at[pl.ds(base, TPW)], w1_v)
        pltpu.sync_copy(w2_hbm.at[pl.ds(base, TPW)], w2_v)
        for c in range(NCHUNK):
            sidx[2 * c] = p1_v[pl.ds(c * CH, CH)]
            sidx[2 * c + 1] = p2_v[pl.ds(c * CH, CH)]
            ca = pltpu.async_copy(
                res_hbm.at[sidx.at[2 * c]], buf_a, sem)
            cb = pltpu.async_copy(
                res_hbm.at[sidx.at[2 * c + 1]], buf_b, sem)
            ca.wait()
            cb.wait()
            def rbody(r, _):
                tsel = jnp.full((16,), c * CH, jnp.int32) + r
                wa = plsc.load_gather(w1_v, [tsel])
                wb = plsc.load_gather(w2_v, [tsel])

                def fbody(f, _):
                    a = buf_a[r, pl.ds(f * 16, 16)]
                    b = buf_b[r, pl.ds(f * 16, 16)]
                    buf_o[r, pl.ds(f * 16, 16)] = wa * a + wb * b
                    return 0

                lax.fori_loop(0, D // 16, fbody, 0)
                return 0

            lax.fori_loop(0, CH, rbody, 0)
            pltpu.sync_copy(buf_o, out_hbm.at[pl.ds(base + c * CH, CH)])

    return k(res, p1, p2, w1, w2)


def _expert_body(wi_b, wi_e, wi_valid, offs, xs_ref, w1_ref, w2_ref, out_ref):
    i = pl.program_id(0)
    b = wi_b[i]
    e = wi_e[i]
    first = jnp.logical_or(i == 0, wi_b[jnp.maximum(i - 1, 0)] != b)

    @pl.when(first)
    def _():
        out_ref[...] = jnp.zeros_like(out_ref)

    @pl.when(wi_valid[i] == 1)
    def _():
        start = offs[e]
        end = offs[e + 1]
        gid = b * GB + lax.broadcasted_iota(jnp.int32, (GB, 1), 0)
        msk = (gid >= start) & (gid < end)
        xm = jnp.where(msk, xs_ref[...], 0.0).astype(jnp.bfloat16)
        h = jnp.dot(xm, w1_ref[0], preferred_element_type=jnp.float32)
        h = (h * jax.nn.sigmoid(h)).astype(jnp.bfloat16)
        y = jnp.dot(h, w2_ref[0], preferred_element_type=jnp.float32)
        out_ref[...] += y


def _run_experts(xs, w1b, w2b, wi_b, wi_e, wi_valid, offs):
    grid_spec = pltpu.PrefetchScalarGridSpec(
        num_scalar_prefetch=4,
        grid=(NI,),
        in_specs=[
            pl.BlockSpec((GB, D), lambda i, b, e, v, o: (b[i], 0)),
            pl.BlockSpec((1, D, FF), lambda i, b, e, v, o: (e[i], 0, 0)),
            pl.BlockSpec((1, FF, D), lambda i, b, e, v, o: (e[i], 0, 0)),
        ],
        out_specs=pl.BlockSpec((GB, D), lambda i, b, e, v, o: (b[i], 0)),
    )
    return pl.pallas_call(
        _expert_body,
        grid_spec=grid_spec,
        out_shape=jax.ShapeDtypeStruct((NROWS, D), jnp.float32),
        compiler_params=pltpu.CompilerParams(
            dimension_semantics=("arbitrary",)),
        interpret=_INTERPRET,
    )(wi_b, wi_e, wi_valid, offs, xs, w1b, w2b)


def kernel(input, Wg, W1, W2):
    x = input
    wg_pad = jnp.zeros((D, 128), jnp.float32).at[:, :NUM_E].set(Wg)
    e1, e2, r1, r2, w1, w2, cnt = _run_router(x, wg_pad)
    e1 = e1[:, 0]
    e2 = e2[:, 0]
    r1 = r1[:, 0]
    r2 = r2[:, 0]
    w1 = w1[:, 0]
    w2 = w2[:, 0]
    counts = cnt[0, :NUM_E].astype(jnp.int32)

    offs = jnp.concatenate([jnp.zeros((1,), jnp.int32),
                            jnp.cumsum(counts)]).astype(jnp.int32)
    bi = jnp.arange(NB, dtype=jnp.int32)[:, None]
    ei = jnp.arange(NUM_E, dtype=jnp.int32)[None, :]
    start = offs[:NUM_E][None, :]
    end = offs[1:][None, :]
    active = (start < (bi + 1) * GB) & (end > bi * GB)
    aflat = active.reshape(-1)
    bflat = jnp.broadcast_to(bi, (NB, NUM_E)).reshape(-1)
    eflat = jnp.broadcast_to(ei, (NB, NUM_E)).reshape(-1)
    item_pos = jnp.cumsum(aflat.astype(jnp.int32)) - aflat.astype(jnp.int32)
    idx = jnp.where(aflat, item_pos, NI)
    n_active = jnp.sum(aflat.astype(jnp.int32))
    wi_b = jnp.full((NI,), NB - 1, jnp.int32).at[idx].set(bflat, mode="drop")
    wi_e0 = jnp.zeros((NI,), jnp.int32).at[idx].set(eflat, mode="drop")
    last_e = wi_e0[jnp.maximum(n_active - 1, 0)]
    tail = jnp.arange(NI, dtype=jnp.int32) >= n_active
    wi_e = jnp.where(tail, last_e, wi_e0)
    wi_valid = jnp.where(tail, 0, 1).astype(jnp.int32)

    off16 = jnp.zeros((16,), jnp.int32).at[:NUM_E + 1].set(offs)
    xs, pos1, pos2 = _dispatch_sc(x, e1, e2, r1, r2, off16)

    w1b = W1.astype(jnp.bfloat16)
    w2b = W2.astype(jnp.bfloat16)
    res = _run_experts(xs, w1b, w2b, wi_b, wi_e, wi_valid, offs)

    out = _combine_sc(res, pos1, pos2, w1, w2)
    return out


# speedup vs baseline: 1.3097x; 1.0132x over previous
import functools

import jax
import jax.numpy as jnp
from jax import lax
from jax.experimental import pallas as pl
from jax.experimental.pallas import tpu as pltpu
from jax.experimental.pallas import tpu_sc as plsc

NUM_E = 8
TOPK = 2
T = 2048
D = 2048
FF = 2048
TB = 512
GB = 256
NROWS = T * TOPK
NB = NROWS // GB
NI = NB + NUM_E
NROWS_PAD = NI * GB

_INTERPRET = False


def _router_body(x_ref, wg_ref, e1_ref, e2_ref, r1_ref, r2_ref,
                 w1_ref, w2_ref, cnt_ref, carry):
    i = pl.program_id(0)

    @pl.when(i == 0)
    def _():
        carry[...] = jnp.zeros_like(carry)

    x = x_ref[...]
    wg = wg_ref[...]
    logits = jnp.dot(x, wg, preferred_element_type=jnp.float32)
    lane = lax.broadcasted_iota(jnp.int32, (TB, 128), 1)
    valid = lane < NUM_E
    neg = jnp.float32(-1e30)
    lv = jnp.where(valid, logits, neg)
    m1 = jnp.max(lv, axis=1, keepdims=True)
    e1 = jnp.min(jnp.where((lv == m1) & valid, lane, 127), axis=1,
                 keepdims=True)
    lv2 = jnp.where(lane == e1, neg, lv)
    m2 = jnp.max(lv2, axis=1, keepdims=True)
    e2 = jnp.min(jnp.where((lv2 == m2) & valid, lane, 127), axis=1,
                 keepdims=True)
    w1 = jax.nn.sigmoid(m1 - m2)
    w2 = 1.0 - w1

    oh1 = (lane == e1).astype(jnp.float32)
    oh2 = (lane == e2).astype(jnp.float32)
    ri = lax.broadcasted_iota(jnp.int32, (TB, TB), 0)
    ci = lax.broadcasted_iota(jnp.int32, (TB, TB), 1)
    strict_lower = (ci < ri).astype(jnp.float32)
    c1 = jnp.dot(strict_lower, oh1, preferred_element_type=jnp.float32)
    c2 = (jnp.dot(strict_lower, oh2, preferred_element_type=jnp.float32)
          + jnp.sum(oh1, axis=0, keepdims=True))
    carry_v = carry[...]
    r1 = jnp.sum((c1 + carry_v) * oh1, axis=1, keepdims=True)
    r2 = jnp.sum((c2 + carry_v) * oh2, axis=1, keepdims=True)
    carry[...] = carry_v + jnp.sum(oh1 + oh2, axis=0, keepdims=True)

    e1_ref[...] = e1
    e2_ref[...] = e2
    r1_ref[...] = r1.astype(jnp.int32)
    r2_ref[...] = r2.astype(jnp.int32)
    w1_ref[...] = w1
    w2_ref[...] = w2
    cnt_ref[...] = carry[...]


def _run_router(x, wg_pad):
    col = lambda dt: jax.ShapeDtypeStruct((T, 1), dt)
    outs = [col(jnp.int32), col(jnp.int32), col(jnp.int32), col(jnp.int32),
            col(jnp.float32), col(jnp.float32),
            jax.ShapeDtypeStruct((1, 128), jnp.float32)]
    colspec = pl.BlockSpec((TB, 1), lambda i: (i, 0))
    return pl.pallas_call(
        _router_body,
        grid=(T // TB,),
        in_specs=[pl.BlockSpec((TB, D), lambda i: (i, 0)),
                  pl.BlockSpec((D, 128), lambda i: (0, 0))],
        out_specs=[colspec] * 6 + [pl.BlockSpec((1, 128), lambda i: (0, 0))],
        out_shape=outs,
        scratch_shapes=[pltpu.VMEM((1, 128), jnp.float32)],
        interpret=_INTERPRET,
    )(x, wg_pad)


_SC_NC = 2
_SC_NS = 16
_SC_NW = _SC_NC * _SC_NS
TPW = T // _SC_NW
CH = 16
NCHUNK = TPW // CH


def _dispatch_sc(x, e1, e2, r1, r2, off16):
    @functools.partial(
        pl.kernel,
        mesh=plsc.VectorSubcoreMesh(core_axis_name="c", subcore_axis_name="s"),
        out_type=[jax.ShapeDtypeStruct((NROWS_PAD, D), jnp.float32),
                  jax.ShapeDtypeStruct((T,), jnp.int32),
                  jax.ShapeDtypeStruct((T,), jnp.int32)],
        compiler_params=pltpu.CompilerParams(needs_layout_passes=False),
        scratch_types=[
            pltpu.VMEM((16,), jnp.int32),
            pltpu.VMEM((TPW,), jnp.int32),
            pltpu.VMEM((TPW,), jnp.int32),
            pltpu.VMEM((TPW,), jnp.int32),
            pltpu.VMEM((TPW,), jnp.int32),
            pltpu.VMEM((TPW,), jnp.int32),
            pltpu.VMEM((TPW,), jnp.int32),
            pltpu.VMEM((2 * NCHUNK, CH), jnp.int32),
            pltpu.VMEM((CH, D), jnp.float32),
            pltpu.SemaphoreType.DMA,
        ],
    )
    def k(x_hbm, e1_hbm, e2_hbm, r1_hbm, r2_hbm, off_hbm,
          xs_hbm, p1_hbm, p2_hbm,
          off_v, e1_v, e2_v, r1_v, r2_v, p1_v, p2_v, sidx, rows_v,
          sem):
        wid = lax.axis_index("s") * _SC_NC + lax.axis_index("c")
        base = wid * TPW
        pltpu.sync_copy(off_hbm, off_v)
        pltpu.sync_copy(e1_hbm.at[pl.ds(base, TPW)], e1_v)
        pltpu.sync_copy(e2_hbm.at[pl.ds(base, TPW)], e2_v)
        pltpu.sync_copy(r1_hbm.at[pl.ds(base, TPW)], r1_v)
        pltpu.sync_copy(r2_hbm.at[pl.ds(base, TPW)], r2_v)
        for c in range(NCHUNK):
            p1 = plsc.load_gather(off_v, [e1_v[pl.ds(c * CH, CH)]]) \
                + r1_v[pl.ds(c * CH, CH)]
            p2 = plsc.load_gather(off_v, [e2_v[pl.ds(c * CH, CH)]]) \
                + r2_v[pl.ds(c * CH, CH)]
            p1_v[pl.ds(c * CH, CH)] = p1
            p2_v[pl.ds(c * CH, CH)] = p2
            sidx[2 * c] = p1
            sidx[2 * c + 1] = p2
            pltpu.sync_copy(x_hbm.at[pl.ds(base + c * CH, CH)], rows_v)
            cp1 = pltpu.async_copy(rows_v, xs_hbm.at[sidx.at[2 * c]], sem)
            cp2 = pltpu.async_copy(rows_v, xs_hbm.at[sidx.at[2 * c + 1]], sem)
            cp1.wait()
            cp2.wait()
        pltpu.sync_copy(p1_v, p1_hbm.at[pl.ds(base, TPW)])
        pltpu.sync_copy(p2_v, p2_hbm.at[pl.ds(base, TPW)])

    return k(x, e1, e2, r1, r2, off16)


def _combine_sc(res, p1, p2, w1, w2):
    @functools.partial(
        pl.kernel,
        mesh=plsc.VectorSubcoreMesh(core_axis_name="c", subcore_axis_name="s"),
        out_type=jax.ShapeDtypeStruct((T, D), jnp.float32),
        compiler_params=pltpu.CompilerParams(needs_layout_passes=False),
        scratch_types=[
            pltpu.VMEM((TPW,), jnp.int32),
            pltpu.VMEM((TPW,), jnp.int32),
            pltpu.VMEM((TPW,), jnp.float32),
            pltpu.VMEM((TPW,), jnp.float32),
            pltpu.VMEM((2 * NCHUNK, CH), jnp.int32),
            pltpu.VMEM((CH, D), jnp.float32),
            pltpu.VMEM((CH, D), jnp.float32),
            pltpu.VMEM((CH, D), jnp.float32),
            pltpu.SemaphoreType.DMA,
        ],
    )
    def k(res_hbm, p1_hbm, p2_hbm, w1_hbm, w2_hbm, out_hbm,
          p1_v, p2_v, w1_v, w2_v, sidx, buf_a, buf_b, buf_o, sem):
        wid = lax.axis_index("s") * _SC_NC + lax.axis_index("c")
        base = wid * TPW
        pltpu.sync_copy(p1_hbm.at[pl.ds(base, TPW)], p1_v)
        pltpu.sync_copy(p2_hbm.at[pl.ds(base, TPW)], p2_v)
        pltpu.sync_copy(w1_hbm.at[pl.ds(base, TPW)], w1_v)
        pltpu.sync_copy(w2_hbm.at[pl.ds(base, TPW)], w2_v)
        for c in range(NCHUNK):
            sidx[2 * c] = p1_v[pl.ds(c * CH, CH)]
            sidx[2 * c + 1] = p2_v[pl.ds(c * CH, CH)]
            ca = pltpu.async_copy(
                res_hbm.at[sidx.at[2 * c]], buf_a, sem)
            cb = pltpu.async_copy(
                res_hbm.at[sidx.at[2 * c + 1]], buf_b, sem)
            ca.wait()
            cb.wait()
            def rbody(r, _):
                tsel = jnp.full((16,), c * CH, jnp.int32) + r
                wa = plsc.load_gather(w1_v, [tsel])
                wb = plsc.load_gather(w2_v, [tsel])

                def fbody(f, _):
                    a = buf_a[r, pl.ds(f * 16, 16)]
                    b = buf_b[r, pl.ds(f * 16, 16)]
                    buf_o[r, pl.ds(f * 16, 16)] = wa * a + wb * b
                    return 0

                lax.fori_loop(0, D // 16, fbody, 0)
                return 0

            lax.fori_loop(0, CH, rbody, 0)
            pltpu.sync_copy(buf_o, out_hbm.at[pl.ds(base + c * CH, CH)])

    return k(res, p1, p2, w1, w2)


def _expert_body(wi_b, wi_e, wi_valid, xs_ref, w1_ref, w2_ref, out_ref):
    i = pl.program_id(0)

    @pl.when(wi_valid[i] == 1)
    def _():
        xm = xs_ref[...].astype(jnp.bfloat16)
        h = jnp.dot(xm, w1_ref[0], preferred_element_type=jnp.float32)
        h = (h * jax.nn.sigmoid(h)).astype(jnp.bfloat16)
        out_ref[...] = jnp.dot(h, w2_ref[0], preferred_element_type=jnp.float32)


def _run_experts(xs, w1b, w2b, wi_b, wi_e, wi_valid):
    grid_spec = pltpu.PrefetchScalarGridSpec(
        num_scalar_prefetch=3,
        grid=(NI,),
        in_specs=[
            pl.BlockSpec((GB, D), lambda i, b, e, v: (b[i], 0)),
            pl.BlockSpec((1, D, FF), lambda i, b, e, v: (e[i], 0, 0)),
            pl.BlockSpec((1, FF, D), lambda i, b, e, v: (e[i], 0, 0)),
        ],
        out_specs=pl.BlockSpec((GB, D), lambda i, b, e, v: (b[i], 0)),
    )
    return pl.pallas_call(
        _expert_body,
        grid_spec=grid_spec,
        out_shape=jax.ShapeDtypeStruct((NROWS_PAD, D), jnp.float32),
        compiler_params=pltpu.CompilerParams(
            dimension_semantics=("arbitrary",)),
        interpret=_INTERPRET,
    )(wi_b, wi_e, wi_valid, xs, w1b, w2b)


def kernel(input, Wg, W1, W2):
    x = input
    wg_pad = jnp.zeros((D, 128), jnp.float32).at[:, :NUM_E].set(Wg)
    e1, e2, r1, r2, w1, w2, cnt = _run_router(x, wg_pad)
    e1 = e1[:, 0]
    e2 = e2[:, 0]
    r1 = r1[:, 0]
    r2 = r2[:, 0]
    w1 = w1[:, 0]
    w2 = w2[:, 0]
    counts = cnt[0, :NUM_E].astype(jnp.int32)

    blocks_per_e = (counts + (GB - 1)) // GB
    blk_off = jnp.concatenate([jnp.zeros((1,), jnp.int32),
                               jnp.cumsum(blocks_per_e)]).astype(jnp.int32)
    aligned_off = blk_off * GB
    n_active = blk_off[NUM_E]
    gidx = jnp.arange(NI, dtype=jnp.int32)
    wi_b = jnp.minimum(gidx, n_active - 1)
    wi_e = jnp.sum((wi_b[:, None] >= blk_off[None, 1:]).astype(jnp.int32),
                   axis=1)
    wi_valid = (gidx < n_active).astype(jnp.int32)

    off16 = jnp.zeros((16,), jnp.int32).at[:NUM_E + 1].set(aligned_off)
    xs, pos1, pos2 = _dispatch_sc(x, e1, e2, r1, r2, off16)

    w1b = W1.astype(jnp.bfloat16)
    w2b = W2.astype(jnp.bfloat16)
    res = _run_experts(xs, w1b, w2b, wi_b, wi_e, wi_valid)

    out = _combine_sc(res, pos1, pos2, w1, w2)
    return out


# speedup vs baseline: 1.4805x; 1.1304x over previous
import functools

import jax
import jax.numpy as jnp
from jax import lax
from jax.experimental import pallas as pl
from jax.experimental.pallas import tpu as pltpu
from jax.experimental.pallas import tpu_sc as plsc

NUM_E = 8
TOPK = 2
T = 2048
D = 2048
FF = 2048
TB = 512
GB = 256
NROWS = T * TOPK
NB = NROWS // GB
NI = NB + NUM_E
NROWS_PAD = NI * GB

_INTERPRET = False


def _router_body(x_ref, wg_ref, e1_ref, e2_ref, r1_ref, r2_ref,
                 w1_ref, w2_ref, cnt_ref, carry):
    i = pl.program_id(0)

    @pl.when(i == 0)
    def _():
        carry[...] = jnp.zeros_like(carry)

    x = x_ref[...]
    wg = wg_ref[...]
    logits = jnp.dot(x, wg, preferred_element_type=jnp.float32)
    lane = lax.broadcasted_iota(jnp.int32, (TB, 128), 1)
    valid = lane < NUM_E
    neg = jnp.float32(-1e30)
    lv = jnp.where(valid, logits, neg)
    m1 = jnp.max(lv, axis=1, keepdims=True)
    e1 = jnp.min(jnp.where((lv == m1) & valid, lane, 127), axis=1,
                 keepdims=True)
    lv2 = jnp.where(lane == e1, neg, lv)
    m2 = jnp.max(lv2, axis=1, keepdims=True)
    e2 = jnp.min(jnp.where((lv2 == m2) & valid, lane, 127), axis=1,
                 keepdims=True)
    w1 = jax.nn.sigmoid(m1 - m2)
    w2 = 1.0 - w1

    oh1 = (lane == e1).astype(jnp.float32)
    oh2 = (lane == e2).astype(jnp.float32)
    ri = lax.broadcasted_iota(jnp.int32, (TB, TB), 0)
    ci = lax.broadcasted_iota(jnp.int32, (TB, TB), 1)
    strict_lower = (ci < ri).astype(jnp.float32)
    c1 = jnp.dot(strict_lower, oh1, preferred_element_type=jnp.float32)
    c2 = (jnp.dot(strict_lower, oh2, preferred_element_type=jnp.float32)
          + jnp.sum(oh1, axis=0, keepdims=True))
    carry_v = carry[...]
    r1 = jnp.sum((c1 + carry_v) * oh1, axis=1, keepdims=True)
    r2 = jnp.sum((c2 + carry_v) * oh2, axis=1, keepdims=True)
    carry[...] = carry_v + jnp.sum(oh1 + oh2, axis=0, keepdims=True)

    e1_ref[...] = e1
    e2_ref[...] = e2
    r1_ref[...] = r1.astype(jnp.int32)
    r2_ref[...] = r2.astype(jnp.int32)
    w1_ref[...] = w1
    w2_ref[...] = w2
    cnt_ref[...] = carry[...]


def _run_router(x, wg_pad):
    col = lambda dt: jax.ShapeDtypeStruct((T, 1), dt)
    outs = [col(jnp.int32), col(jnp.int32), col(jnp.int32), col(jnp.int32),
            col(jnp.float32), col(jnp.float32),
            jax.ShapeDtypeStruct((1, 128), jnp.float32)]
    colspec = pl.BlockSpec((TB, 1), lambda i: (i, 0))
    return pl.pallas_call(
        _router_body,
        grid=(T // TB,),
        in_specs=[pl.BlockSpec((TB, D), lambda i: (i, 0)),
                  pl.BlockSpec((D, 128), lambda i: (0, 0))],
        out_specs=[colspec] * 6 + [pl.BlockSpec((1, 128), lambda i: (0, 0))],
        out_shape=outs,
        scratch_shapes=[pltpu.VMEM((1, 128), jnp.float32)],
        interpret=_INTERPRET,
    )(x, wg_pad)


_SC_NC = 2
_SC_NS = 16
_SC_NW = _SC_NC * _SC_NS
TPW = T // _SC_NW
CH = 16
NCHUNK = TPW // CH


def _dispatch_sc(x, e1, e2, r1, r2, off16):
    @functools.partial(
        pl.kernel,
        mesh=plsc.VectorSubcoreMesh(core_axis_name="c", subcore_axis_name="s"),
        out_type=[jax.ShapeDtypeStruct((NROWS_PAD, D), jnp.float32),
                  jax.ShapeDtypeStruct((T,), jnp.int32),
                  jax.ShapeDtypeStruct((T,), jnp.int32)],
        compiler_params=pltpu.CompilerParams(needs_layout_passes=False),
        scratch_types=[
            pltpu.VMEM((16,), jnp.int32),
            pltpu.VMEM((TPW,), jnp.int32),
            pltpu.VMEM((TPW,), jnp.int32),
            pltpu.VMEM((TPW,), jnp.int32),
            pltpu.VMEM((TPW,), jnp.int32),
            pltpu.VMEM((TPW,), jnp.int32),
            pltpu.VMEM((TPW,), jnp.int32),
            pltpu.VMEM((2 * NCHUNK, CH), jnp.int32),
            pltpu.VMEM((CH, D), jnp.float32),
            pltpu.SemaphoreType.DMA,
        ],
    )
    def k(x_hbm, e1_hbm, e2_hbm, r1_hbm, r2_hbm, off_hbm,
          xs_hbm, p1_hbm, p2_hbm,
          off_v, e1_v, e2_v, r1_v, r2_v, p1_v, p2_v, sidx, rows_v,
          sem):
        wid = lax.axis_index("s") * _SC_NC + lax.axis_index("c")
        base = wid * TPW
        pltpu.sync_copy(off_hbm, off_v)
        pltpu.sync_copy(e1_hbm.at[pl.ds(base, TPW)], e1_v)
        pltpu.sync_copy(e2_hbm.at[pl.ds(base, TPW)], e2_v)
        pltpu.sync_copy(r1_hbm.at[pl.ds(base, TPW)], r1_v)
        pltpu.sync_copy(r2_hbm.at[pl.ds(base, TPW)], r2_v)
        for c in range(NCHUNK):
            p1 = plsc.load_gather(off_v, [e1_v[pl.ds(c * CH, CH)]]) \
                + r1_v[pl.ds(c * CH, CH)]
            p2 = plsc.load_gather(off_v, [e2_v[pl.ds(c * CH, CH)]]) \
                + r2_v[pl.ds(c * CH, CH)]
            p1_v[pl.ds(c * CH, CH)] = p1
            p2_v[pl.ds(c * CH, CH)] = p2
            sidx[2 * c] = p1
            sidx[2 * c + 1] = p2
            pltpu.sync_copy(x_hbm.at[pl.ds(base + c * CH, CH)], rows_v)
            cp1 = pltpu.async_copy(rows_v, xs_hbm.at[sidx.at[2 * c]], sem)
            cp2 = pltpu.async_copy(rows_v, xs_hbm.at[sidx.at[2 * c + 1]], sem)
            cp1.wait()
            cp2.wait()
        pltpu.sync_copy(p1_v, p1_hbm.at[pl.ds(base, TPW)])
        pltpu.sync_copy(p2_v, p2_hbm.at[pl.ds(base, TPW)])

    return k(x, e1, e2, r1, r2, off16)


def _combine_sc(res, p1, p2, w1, w2):
    @functools.partial(
        pl.kernel,
        mesh=plsc.VectorSubcoreMesh(core_axis_name="c", subcore_axis_name="s"),
        out_type=jax.ShapeDtypeStruct((T, D), jnp.float32),
        compiler_params=pltpu.CompilerParams(needs_layout_passes=False),
        scratch_types=[
            pltpu.VMEM((TPW,), jnp.int32),
            pltpu.VMEM((TPW,), jnp.int32),
            pltpu.VMEM((TPW,), jnp.float32),
            pltpu.VMEM((TPW,), jnp.float32),
            pltpu.VMEM((2 * NCHUNK, CH), jnp.int32),
            pltpu.VMEM((CH, D), jnp.float32),
            pltpu.VMEM((CH, D), jnp.float32),
            pltpu.VMEM((CH, D), jnp.float32),
            pltpu.SemaphoreType.DMA,
        ],
    )
    def k(res_hbm, p1_hbm, p2_hbm, w1_hbm, w2_hbm, out_hbm,
          p1_v, p2_v, w1_v, w2_v, sidx, buf_a, buf_b, buf_o, sem):
        wid = lax.axis_index("s") * _SC_NC + lax.axis_index("c")
        base = wid * TPW
        pltpu.sync_copy(p1_hbm.at[pl.ds(base, TPW)], p1_v)
        pltpu.sync_copy(p2_hbm.at[pl.ds(base, TPW)], p2_v)
        pltpu.sync_copy(w1_hbm.at[pl.ds(base, TPW)], w1_v)
        pltpu.sync_copy(w2_hbm.at[pl.ds(base, TPW)], w2_v)
        for c in range(NCHUNK):
            sidx[2 * c] = p1_v[pl.ds(c * CH, CH)]
            sidx[2 * c + 1] = p2_v[pl.ds(c * CH, CH)]
            ca = pltpu.async_copy(
                res_hbm.at[sidx.at[2 * c]], buf_a, sem)
            cb = pltpu.async_copy(
                res_hbm.at[sidx.at[2 * c + 1]], buf_b, sem)
            ca.wait()
            cb.wait()
            def rbody(r, _):
                tsel = jnp.full((16,), c * CH, jnp.int32) + r
                wa = plsc.load_gather(w1_v, [tsel])
                wb = plsc.load_gather(w2_v, [tsel])

                def fbody(f, _):
                    a = buf_a[r, pl.ds(f * 16, 16)]
                    b = buf_b[r, pl.ds(f * 16, 16)]
                    buf_o[r, pl.ds(f * 16, 16)] = wa * a + wb * b
                    return 0

                lax.fori_loop(0, D // 16, fbody, 0)
                return 0

            lax.fori_loop(0, CH, rbody, 0)
            pltpu.sync_copy(buf_o, out_hbm.at[pl.ds(base + c * CH, CH)])

    return k(res, p1, p2, w1, w2)


def _expert_body(wi_b, wi_e, wi_valid, xs_ref, w1_ref, w2_ref, out_ref):
    i = pl.program_id(0)

    @pl.when(wi_valid[i] == 1)
    def _():
        xm = xs_ref[...].astype(jnp.bfloat16)
        w1 = w1_ref[0].astype(jnp.bfloat16)
        h = jnp.dot(xm, w1, preferred_element_type=jnp.float32)
        h = (h * jax.nn.sigmoid(h)).astype(jnp.bfloat16)
        out_ref[...] = jnp.dot(h, w2_ref[0], preferred_element_type=jnp.float32)


def _run_experts(xs, w1b, w2b, wi_b, wi_e, wi_valid):
    grid_spec = pltpu.PrefetchScalarGridSpec(
        num_scalar_prefetch=3,
        grid=(NI,),
        in_specs=[
            pl.BlockSpec((GB, D), lambda i, b, e, v: (b[i], 0)),
            pl.BlockSpec((1, D, FF), lambda i, b, e, v: (e[i], 0, 0)),
            pl.BlockSpec((1, FF, D), lambda i, b, e, v: (e[i], 0, 0)),
        ],
        out_specs=pl.BlockSpec((GB, D), lambda i, b, e, v: (b[i], 0)),
    )
    return pl.pallas_call(
        _expert_body,
        grid_spec=grid_spec,
        out_shape=jax.ShapeDtypeStruct((NROWS_PAD, D), jnp.float32),
        compiler_params=pltpu.CompilerParams(
            dimension_semantics=("arbitrary",)),
        interpret=_INTERPRET,
    )(wi_b, wi_e, wi_valid, xs, w1b, w2b)


def kernel(input, Wg, W1, W2):
    x = input
    wg_pad = jnp.zeros((D, 128), jnp.float32).at[:, :NUM_E].set(Wg)
    e1, e2, r1, r2, w1, w2, cnt = _run_router(x, wg_pad)
    e1 = e1[:, 0]
    e2 = e2[:, 0]
    r1 = r1[:, 0]
    r2 = r2[:, 0]
    w1 = w1[:, 0]
    w2 = w2[:, 0]
    counts = cnt[0, :NUM_E].astype(jnp.int32)

    blocks_per_e = (counts + (GB - 1)) // GB
    blk_off = jnp.concatenate([jnp.zeros((1,), jnp.int32),
                               jnp.cumsum(blocks_per_e)]).astype(jnp.int32)
    aligned_off = blk_off * GB
    n_active = blk_off[NUM_E]
    gidx = jnp.arange(NI, dtype=jnp.int32)
    wi_b = jnp.minimum(gidx, n_active - 1)
    wi_e = jnp.sum((wi_b[:, None] >= blk_off[None, 1:]).astype(jnp.int32),
                   axis=1)
    wi_valid = (gidx < n_active).astype(jnp.int32)

    off16 = jnp.zeros((16,), jnp.int32).at[:NUM_E + 1].set(aligned_off)
    xs, pos1, pos2 = _dispatch_sc(x, e1, e2, r1, r2, off16)

    res = _run_experts(xs, W1, W2.astype(jnp.bfloat16), wi_b, wi_e, wi_valid)

    out = _combine_sc(res, pos1, pos2, w1, w2)
    return out
